# baseline (speedup 1.0000x reference)
import jax
import jax.numpy as jnp
from jax import lax
from jax.experimental import pallas as pl
from jax.experimental.pallas import tpu as pltpu
from jax.experimental.pallas import tpu_sc as plsc

N = 10000
LANES = 16
CHUNK = 128
NCORES = 2
NSUB = 16
STRIPE = N // NSUB
S_PAD = 10016
AL_PAD = 10008


def _matmul_attn(z, W, a_s, a_d):
    n, _ = z.shape
    c = W.shape[1]
    hw = c // 2

    def body(z_ref, w_ref, as_ref, ad_ref, h0_ref, h1_ref, al_ref):
        h = jnp.dot(z_ref[...], w_ref[...], preferred_element_type=jnp.float32)
        h0_ref[...] = h[:, :hw]
        h1_ref[...] = h[:, hw:]
        al_s = jnp.dot(h, as_ref[0], preferred_element_type=jnp.float32)
        al_d = jnp.dot(h, ad_ref[0], preferred_element_type=jnp.float32)
        al = jnp.stack([al_s, al_d])
        al_ref[...] = jnp.concatenate(
            [al, jnp.zeros((2, AL_PAD - n), jnp.float32)], axis=1)

    return pl.pallas_call(
        body,
        out_shape=[
            jax.ShapeDtypeStruct((n, hw), jnp.float32),
            jax.ShapeDtypeStruct((n, hw), jnp.float32),
            jax.ShapeDtypeStruct((2, AL_PAD), jnp.float32),
        ],
    )(z, W, a_s.reshape(1, c), a_d.reshape(1, c))


def _combine_bn(parts, s_parts, b, g, be):
    c = b.shape[0]

    def body(p_ref, s_ref, b_ref, g_ref, be_ref, z_ref):
        hsum = jnp.concatenate([p_ref[0], p_ref[1]], axis=1)
        s = jnp.sum(s_ref[...], axis=1, keepdims=True)
        z = hsum / (s + 1e-16) + b_ref[0][None, :]
        mean = jnp.mean(z, axis=0, keepdims=True)
        var = jnp.mean((z - mean) ** 2, axis=0, keepdims=True)
        xn = (z - mean) * lax.rsqrt(var + 1e-5)
        z_ref[...] = jnp.maximum(xn * g_ref[0][None, :] + be_ref[0][None, :], 0.0)

    return pl.pallas_call(
        body,
        out_shape=jax.ShapeDtypeStruct((N, c), jnp.float32),
    )(parts, s_parts, b.reshape(1, c), g.reshape(1, c), be.reshape(1, c))


def _sc_gat(h0, h1, al, src3d, dst3d, nc, etot):
    hw = h0.shape[1]
    gv = hw // LANES
    mesh = plsc.VectorSubcoreMesh(
        core_axis_name="c", subcore_axis_name="s",
        num_cores=NCORES, num_subcores=NSUB)

    def body(h0_hbm, h1_hbm, al_hbm, src_hbm, dst_hbm, out_hbm, s_hbm,
             als_v, ald_v, srcw, dstw, wj, ssb, rows0, rows1,
             out_sh, s_sh, sem0, sem1):
        cid = lax.axis_index("c")
        sid = lax.axis_index("s")
        off = sid * STRIPE

        zero16 = jnp.zeros((LANES,), jnp.float32)

        def zs(i, _):
            ssb[i, pl.ds(0, LANES)] = zero16
            return 0
        lax.fori_loop(0, CHUNK, zs, 0)
        done = 0
        while done < STRIPE:
            sz = min(CHUNK, STRIPE - done)
            pltpu.sync_copy(ssb.at[pl.ds(0, sz)],
                            s_sh.at[pl.ds(off + done, sz)])
            done += sz

        def zrow(i, _):
            for gidx in range(gv):
                rows0[i, pl.ds(gidx * LANES, LANES)] = zero16
            return 0
        lax.fori_loop(0, CHUNK, zrow, 0)
        done = 0
        while done < STRIPE:
            sz = min(CHUNK, STRIPE - done)
            pltpu.sync_copy(rows0.at[pl.ds(0, sz)],
                            out_sh.at[pl.ds(off + done, sz)])
            done += sz

        pltpu.sync_copy(al_hbm.at[0], als_v)
        pltpu.sync_copy(al_hbm.at[1], ald_v)
        pltpu.sync_copy(src_hbm.at[sid], srcw)
        pltpu.sync_copy(dst_hbm.at[sid], dstw)

        plsc.subcore_barrier()

        lane = lax.iota(jnp.int32, LANES)
        base0 = sid * (nc * CHUNK)

        def edge_phase(h_hbm, with_s):
            def phase_a(j):
                base1 = base0 + j * CHUNK
                for i in range(CHUNK // LANES):
                    d16 = dstw[j, pl.ds(i * LANES, LANES)]
                    als = plsc.load_gather(
                        als_v, [srcw[j, pl.ds(i * LANES, LANES)]])
                    ald = plsc.load_gather(ald_v, [d16])
                    e = als + ald
                    e = jnp.where(e > 0.0, e, 0.2 * e)
                    w = jnp.exp(e)
                    geid = lax.broadcast(base1 + i * LANES, (LANES,)) + lane
                    w = jnp.where(geid < etot, w, 0.0)
                    wj[pl.ds(i * LANES, LANES)] = w
                    if with_s:
                        plsc.store_scatter(
                            ssb,
                            [lax.broadcast(i * LANES, (LANES,)) + lane,
                             jnp.zeros((LANES,), jnp.int32)], w)
                if with_s:
                    pltpu.sync_copy(ssb, s_sh.at[dstw.at[j]], add=True)

            def mul_scatter(j, rows):
                def mbody(i, _):
                    for u in range(4):
                        idx = i * 4 + u
                        ws = plsc.load_gather(
                            wj, [lax.broadcast(idx, (LANES,))])
                        for gidx in range(gv):
                            sl = pl.ds(gidx * LANES, LANES)
                            rows[idx, sl] = rows[idx, sl] * ws
                    return 0
                lax.fori_loop(0, CHUNK // 4, mbody, 0)
                pltpu.sync_copy(rows, out_sh.at[dstw.at[j]], add=True)

            def half_step(j, rows, sem, pf_j, pf_rows, pf_sem):
                pltpu.async_copy(h_hbm.at[srcw.at[pf_j]], pf_rows, pf_sem)
                phase_a(j)
                pltpu.make_async_copy(
                    h_hbm.at[srcw.at[j]], rows, sem).wait()
                mul_scatter(j, rows)

            pltpu.async_copy(h_hbm.at[srcw.at[0]], rows0, sem0)

            def pair_body(k, _):
                j0 = 2 * k
                j1 = j0 + 1
                j2 = jnp.where(j1 + 1 < nc, j1 + 1, 0)
                half_step(j0, rows0, sem0, j1, rows1, sem1)
                half_step(j1, rows1, sem1, j2, rows0, sem0)
                return 0
            lax.fori_loop(0, nc // 2, pair_body, 0)
            pltpu.make_async_copy(
                h_hbm.at[srcw.at[0]], rows0, sem0).wait()

        @pl.when(cid == 0)
        def _():
            edge_phase(h0_hbm, True)

        @pl.when(cid == 1)
        def _():
            edge_phase(h1_hbm, False)

        plsc.subcore_barrier()

        @pl.when(cid == 0)
        def _():
            pltpu.sync_copy(s_sh.at[pl.ds(off, STRIPE)],
                            s_hbm.at[pl.ds(off, STRIPE)])

        pltpu.sync_copy(out_sh.at[pl.ds(off, STRIPE)],
                        out_hbm.at[cid, pl.ds(off, STRIPE)])

    kern = pl.kernel(
        body,
        out_type=[
            jax.ShapeDtypeStruct((NCORES, N, hw), jnp.float32),
            jax.ShapeDtypeStruct((N, LANES), jnp.float32),
        ],
        mesh=mesh,
        compiler_params=pltpu.CompilerParams(
            needs_layout_passes=False, use_tc_tiling_on_sc=False),
        scratch_types=[
            pltpu.VMEM((AL_PAD,), jnp.float32),
            pltpu.VMEM((AL_PAD,), jnp.float32),
            pltpu.VMEM((nc, CHUNK), jnp.int32),
            pltpu.VMEM((nc, CHUNK), jnp.int32),
            pltpu.VMEM((CHUNK,), jnp.float32),
            pltpu.VMEM((CHUNK, LANES), jnp.float32),
            pltpu.VMEM((CHUNK, hw), jnp.float32),
            pltpu.VMEM((CHUNK, hw), jnp.float32),
            pltpu.VMEM_SHARED((N, hw), jnp.float32),
            pltpu.VMEM_SHARED((N, LANES), jnp.float32),
            pltpu.SemaphoreType.DMA,
            pltpu.SemaphoreType.DMA,
        ],
    )
    return kern(h0, h1, al, src3d, dst3d)


def kernel(x, edge_index, W1, as1, ad1, b1, g1, be1,
           W2, as2, ad2, b2, g2, be2, W3, as3, ad3, b3, g3, be3):
    loops = jnp.arange(N, dtype=jnp.int32)
    E = edge_index.shape[1]
    etot = E + N
    nc = -(-etot // (NSUB * CHUNK))
    epad = NSUB * nc * CHUNK
    pad = epad - etot
    src = jnp.concatenate(
        [edge_index[0], loops, jnp.zeros((pad,), jnp.int32)])
    dst = jnp.concatenate(
        [edge_index[1], loops, jnp.zeros((pad,), jnp.int32)])
    src3d = src.reshape(NSUB, nc, CHUNK)
    dst3d = dst.reshape(NSUB, nc, CHUNK)

    z = x
    for (W, a_s, a_d, b, g, be) in (
            (W1, as1, ad1, b1, g1, be1),
            (W2, as2, ad2, b2, g2, be2),
            (W3, as3, ad3, b3, g3, be3)):
        h0, h1, al = _matmul_attn(z, W, a_s, a_d)
        parts, s_parts = _sc_gat(h0, h1, al, src3d, dst3d, nc, etot)
        z = _combine_bn(parts, s_parts, b, g, be)
    return z


# speedup vs baseline: 1.0084x; 1.0084x over previous
import jax
import jax.numpy as jnp
from jax import lax
from jax.experimental import pallas as pl
from jax.experimental.pallas import tpu as pltpu
from jax.experimental.pallas import tpu_sc as plsc

N = 10000
LANES = 16
CHUNK = 128
NCORES = 2
NSUB = 16
STRIPE = N // NSUB
S_PAD = 10016
AL_PAD = 10008


def _matmul_attn(z, W, a_s, a_d):
    n, _ = z.shape
    c = W.shape[1]
    hw = c // 2

    def body(z_ref, w_ref, as_ref, ad_ref, h0_ref, h1_ref, al_ref):
        h = jnp.dot(z_ref[...].astype(jnp.bfloat16),
                    w_ref[...].astype(jnp.bfloat16),
                    preferred_element_type=jnp.float32)
        h0_ref[...] = h[:, :hw]
        h1_ref[...] = h[:, hw:]
        hb = h.astype(jnp.bfloat16)
        al_s = jnp.dot(hb, as_ref[0].astype(jnp.bfloat16),
                       preferred_element_type=jnp.float32)
        al_d = jnp.dot(hb, ad_ref[0].astype(jnp.bfloat16),
                       preferred_element_type=jnp.float32)
        al = jnp.stack([al_s, al_d])
        al_ref[...] = jnp.concatenate(
            [al, jnp.zeros((2, AL_PAD - n), jnp.float32)], axis=1)

    return pl.pallas_call(
        body,
        out_shape=[
            jax.ShapeDtypeStruct((n, hw), jnp.float32),
            jax.ShapeDtypeStruct((n, hw), jnp.float32),
            jax.ShapeDtypeStruct((2, AL_PAD), jnp.float32),
        ],
    )(z, W, a_s.reshape(1, c), a_d.reshape(1, c))


def _combine_bn(parts, s_parts, b, g, be):
    c = b.shape[0]

    def body(p_ref, s_ref, b_ref, g_ref, be_ref, z_ref):
        hsum = jnp.concatenate([p_ref[0], p_ref[1]], axis=1)
        s = jnp.sum(s_ref[0] + s_ref[1], axis=1, keepdims=True)
        z = hsum / (s + 1e-16) + b_ref[0][None, :]
        mean = jnp.mean(z, axis=0, keepdims=True)
        var = jnp.mean((z - mean) ** 2, axis=0, keepdims=True)
        xn = (z - mean) * lax.rsqrt(var + 1e-5)
        z_ref[...] = jnp.maximum(xn * g_ref[0][None, :] + be_ref[0][None, :], 0.0)

    return pl.pallas_call(
        body,
        out_shape=jax.ShapeDtypeStruct((N, c), jnp.float32),
    )(parts, s_parts, b.reshape(1, c), g.reshape(1, c), be.reshape(1, c))


def _sc_gat(h0, h1, al, src3d, dst3d, nc, etot):
    hw = h0.shape[1]
    gv = hw // LANES
    mesh = plsc.VectorSubcoreMesh(
        core_axis_name="c", subcore_axis_name="s",
        num_cores=NCORES, num_subcores=NSUB)

    def body(h0_hbm, h1_hbm, al_hbm, src_hbm, dst_hbm, out_hbm, s_hbm,
             als_v, ald_v, srcw, dstw, wj, ssb, rows0, rows1,
             out_sh, s_sh, sem0, sem1, sem2, sem3):
        cid = lax.axis_index("c")
        sid = lax.axis_index("s")
        off = sid * STRIPE

        zero16 = jnp.zeros((LANES,), jnp.float32)

        def zs(i, _):
            ssb[i, pl.ds(0, LANES)] = zero16
            return 0
        lax.fori_loop(0, CHUNK, zs, 0)
        done = 0
        while done < STRIPE:
            sz = min(CHUNK, STRIPE - done)
            pltpu.sync_copy(ssb.at[pl.ds(0, sz)],
                            s_sh.at[pl.ds(off + done, sz)])
            done += sz

        def zrow(i, _):
            for gidx in range(gv):
                rows0[i, pl.ds(gidx * LANES, LANES)] = zero16
                rows1[i, pl.ds(gidx * LANES, LANES)] = zero16
            return 0
        lax.fori_loop(0, CHUNK, zrow, 0)
        done = 0
        while done < STRIPE:
            sz = min(CHUNK, STRIPE - done)
            pltpu.sync_copy(rows0.at[pl.ds(0, sz)],
                            out_sh.at[pl.ds(off + done, sz)])
            done += sz

        pltpu.sync_copy(al_hbm.at[0], als_v)
        pltpu.sync_copy(al_hbm.at[1], ald_v)
        pltpu.sync_copy(src_hbm.at[sid], srcw)
        pltpu.sync_copy(dst_hbm.at[sid], dstw)

        plsc.subcore_barrier()

        lane = lax.iota(jnp.int32, LANES)
        base0 = sid * (nc * CHUNK)

        def edge_phase(h_hbm):
            def phase_a(j):
                base1 = base0 + j * CHUNK
                for i in range(CHUNK // LANES):
                    d16 = dstw[j, pl.ds(i * LANES, LANES)]
                    als = plsc.load_gather(
                        als_v, [srcw[j, pl.ds(i * LANES, LANES)]])
                    ald = plsc.load_gather(ald_v, [d16])
                    e = als + ald
                    e = jnp.where(e > 0.0, e, 0.2 * e)
                    w = jnp.exp(e)
                    geid = lax.broadcast(base1 + i * LANES, (LANES,)) + lane
                    w = jnp.where(geid < etot, w, 0.0)
                    wj[pl.ds(i * LANES, LANES)] = w
                    plsc.store_scatter(
                        ssb,
                        [lax.broadcast(i * LANES, (LANES,)) + lane,
                         jnp.zeros((LANES,), jnp.int32)], w)

                @pl.when(lax.rem(j, 2) == cid)
                def _():
                    pltpu.sync_copy(ssb, s_sh.at[dstw.at[j]], add=True)

            def multiply(rows):
                def mbody(i, _):
                    for u in range(4):
                        idx = i * 4 + u
                        ws = plsc.load_gather(
                            wj, [lax.broadcast(idx, (LANES,))])
                        for gidx in range(gv):
                            sl = pl.ds(gidx * LANES, LANES)
                            rows[idx, sl] = rows[idx, sl] * ws
                    return 0
                lax.fori_loop(0, CHUNK // 4, mbody, 0)

            def half_step(j, rows, g_sem, sc_sem, pf_j, pf_rows,
                          pf_g_sem, pf_sc_sem):
                pltpu.make_async_copy(
                    pf_rows, out_sh.at[dstw.at[0]], pf_sc_sem).wait()
                pltpu.async_copy(h_hbm.at[srcw.at[pf_j]], pf_rows, pf_g_sem)
                phase_a(j)
                pltpu.make_async_copy(
                    h_hbm.at[srcw.at[j]], rows, g_sem).wait()
                multiply(rows)
                pltpu.async_copy(rows, out_sh.at[dstw.at[j]], sc_sem,
                                 add=True)

            pltpu.async_copy(h_hbm.at[srcw.at[0]], rows0, sem0)
            pltpu.async_copy(rows1, out_sh.at[dstw.at[0]], sem3, add=True)

            def pair_body(k, _):
                j0 = 2 * k
                j1 = j0 + 1
                j2 = jnp.where(j1 + 1 < nc, j1 + 1, 0)
                half_step(j0, rows0, sem0, sem2, j1, rows1, sem1, sem3)
                half_step(j1, rows1, sem1, sem3, j2, rows0, sem0, sem2)
                return 0
            lax.fori_loop(0, nc // 2, pair_body, 0)
            pltpu.make_async_copy(
                h_hbm.at[srcw.at[0]], rows0, sem0).wait()
            pltpu.make_async_copy(
                rows1, out_sh.at[dstw.at[0]], sem3).wait()

        @pl.when(cid == 0)
        def _():
            edge_phase(h0_hbm)

        @pl.when(cid == 1)
        def _():
            edge_phase(h1_hbm)

        plsc.subcore_barrier()

        pltpu.sync_copy(s_sh.at[pl.ds(off, STRIPE)],
                        s_hbm.at[cid, pl.ds(off, STRIPE)])

        pltpu.sync_copy(out_sh.at[pl.ds(off, STRIPE)],
                        out_hbm.at[cid, pl.ds(off, STRIPE)])

    kern = pl.kernel(
        body,
        out_type=[
            jax.ShapeDtypeStruct((NCORES, N, hw), jnp.float32),
            jax.ShapeDtypeStruct((NCORES, N, LANES), jnp.float32),
        ],
        mesh=mesh,
        compiler_params=pltpu.CompilerParams(
            needs_layout_passes=False, use_tc_tiling_on_sc=False),
        scratch_types=[
            pltpu.VMEM((AL_PAD,), jnp.float32),
            pltpu.VMEM((AL_PAD,), jnp.float32),
            pltpu.VMEM((nc, CHUNK), jnp.int32),
            pltpu.VMEM((nc, CHUNK), jnp.int32),
            pltpu.VMEM((CHUNK,), jnp.float32),
            pltpu.VMEM((CHUNK, LANES), jnp.float32),
            pltpu.VMEM((CHUNK, hw), jnp.float32),
            pltpu.VMEM((CHUNK, hw), jnp.float32),
            pltpu.VMEM_SHARED((N, hw), jnp.float32),
            pltpu.VMEM_SHARED((N, LANES), jnp.float32),
            pltpu.SemaphoreType.DMA,
            pltpu.SemaphoreType.DMA,
            pltpu.SemaphoreType.DMA,
            pltpu.SemaphoreType.DMA,
        ],
    )
    return kern(h0, h1, al, src3d, dst3d)


def kernel(x, edge_index, W1, as1, ad1, b1, g1, be1,
           W2, as2, ad2, b2, g2, be2, W3, as3, ad3, b3, g3, be3):
    loops = jnp.arange(N, dtype=jnp.int32)
    E = edge_index.shape[1]
    etot = E + N
    nc = -(-etot // (NSUB * CHUNK))
    epad = NSUB * nc * CHUNK
    pad = epad - etot
    src = jnp.concatenate(
        [edge_index[0], loops, jnp.zeros((pad,), jnp.int32)])
    dst = jnp.concatenate(
        [edge_index[1], loops, jnp.zeros((pad,), jnp.int32)])
    src3d = src.reshape(NSUB, nc, CHUNK)
    dst3d = dst.reshape(NSUB, nc, CHUNK)

    z = x
    for (W, a_s, a_d, b, g, be) in (
            (W1, as1, ad1, b1, g1, be1),
            (W2, as2, ad2, b2, g2, be2),
            (W3, as3, ad3, b3, g3, be3)):
        h0, h1, al = _matmul_attn(z, W, a_s, a_d)
        parts, s_parts = _sc_gat(h0, h1, al, src3d, dst3d, nc, etot)
        z = _combine_bn(parts, s_parts, b, g, be)
    return z


# speedup vs baseline: 1.0095x; 1.0011x over previous
import jax
import jax.numpy as jnp
from jax import lax
from jax.experimental import pallas as pl
from jax.experimental.pallas import tpu as pltpu
from jax.experimental.pallas import tpu_sc as plsc

N = 10000
LANES = 16
CHUNK = 128
NCORES = 2
NSUB = 16
STRIPE = N // NSUB
S_PAD = 10016
AL_PAD = 10008


def _matmul_attn(z, W, a_s, a_d):
    n, _ = z.shape
    c = W.shape[1]
    hw = c // 2

    def body(z_ref, w_ref, as_ref, ad_ref, h0_ref, h1_ref, al_ref):
        h = jnp.dot(z_ref[...].astype(jnp.bfloat16),
                    w_ref[...].astype(jnp.bfloat16),
                    preferred_element_type=jnp.float32)
        h0_ref[...] = h[:, :hw]
        h1_ref[...] = h[:, hw:]
        hb = h.astype(jnp.bfloat16)
        al_s = jnp.dot(hb, as_ref[0].astype(jnp.bfloat16),
                       preferred_element_type=jnp.float32)
        al_d = jnp.dot(hb, ad_ref[0].astype(jnp.bfloat16),
                       preferred_element_type=jnp.float32)
        al = jnp.stack([al_s, al_d])
        al_ref[...] = jnp.concatenate(
            [al, jnp.zeros((2, AL_PAD - n), jnp.float32)], axis=1)

    return pl.pallas_call(
        body,
        out_shape=[
            jax.ShapeDtypeStruct((n, hw), jnp.float32),
            jax.ShapeDtypeStruct((n, hw), jnp.float32),
            jax.ShapeDtypeStruct((2, AL_PAD), jnp.float32),
        ],
    )(z, W, a_s.reshape(1, c), a_d.reshape(1, c))


def _combine_bn(parts, s_parts, b, g, be):
    c = b.shape[0]

    def body(p_ref, s_ref, b_ref, g_ref, be_ref, z_ref):
        hsum = jnp.concatenate([p_ref[0], p_ref[1]], axis=1)
        s = jnp.sum(s_ref[0] + s_ref[1], axis=1, keepdims=True)
        z = hsum / (s + 1e-16) + b_ref[0][None, :]
        mean = jnp.mean(z, axis=0, keepdims=True)
        var = jnp.mean((z - mean) ** 2, axis=0, keepdims=True)
        xn = (z - mean) * lax.rsqrt(var + 1e-5)
        z_ref[...] = jnp.maximum(xn * g_ref[0][None, :] + be_ref[0][None, :], 0.0)

    return pl.pallas_call(
        body,
        out_shape=jax.ShapeDtypeStruct((N, c), jnp.float32),
    )(parts, s_parts, b.reshape(1, c), g.reshape(1, c), be.reshape(1, c))


def _combine_matmul_attn(parts, s_parts, b, g, be, W, a_s, a_d):
    cp = b.shape[0]
    c = W.shape[1]
    hw = c // 2

    def body(p_ref, s_ref, b_ref, g_ref, be_ref, w_ref, as_ref, ad_ref,
             h0_ref, h1_ref, al_ref):
        hsum = jnp.concatenate([p_ref[0], p_ref[1]], axis=1)
        s = jnp.sum(s_ref[0] + s_ref[1], axis=1, keepdims=True)
        z = hsum / (s + 1e-16) + b_ref[0][None, :]
        mean = jnp.mean(z, axis=0, keepdims=True)
        var = jnp.mean((z - mean) ** 2, axis=0, keepdims=True)
        xn = (z - mean) * lax.rsqrt(var + 1e-5)
        z = jnp.maximum(xn * g_ref[0][None, :] + be_ref[0][None, :], 0.0)
        h = jnp.dot(z.astype(jnp.bfloat16), w_ref[...].astype(jnp.bfloat16),
                    preferred_element_type=jnp.float32)
        h0_ref[...] = h[:, :hw]
        h1_ref[...] = h[:, hw:]
        hb = h.astype(jnp.bfloat16)
        al_s = jnp.dot(hb, as_ref[0].astype(jnp.bfloat16),
                       preferred_element_type=jnp.float32)
        al_d = jnp.dot(hb, ad_ref[0].astype(jnp.bfloat16),
                       preferred_element_type=jnp.float32)
        al = jnp.stack([al_s, al_d])
        al_ref[...] = jnp.concatenate(
            [al, jnp.zeros((2, AL_PAD - N), jnp.float32)], axis=1)

    return pl.pallas_call(
        body,
        out_shape=[
            jax.ShapeDtypeStruct((N, hw), jnp.float32),
            jax.ShapeDtypeStruct((N, hw), jnp.float32),
            jax.ShapeDtypeStruct((2, AL_PAD), jnp.float32),
        ],
    )(parts, s_parts, b.reshape(1, cp), g.reshape(1, cp), be.reshape(1, cp),
      W, a_s.reshape(1, c), a_d.reshape(1, c))


def _sc_gat(h0, h1, al, src3d, dst3d, nc, etot):
    hw = h0.shape[1]
    gv = hw // LANES
    mesh = plsc.VectorSubcoreMesh(
        core_axis_name="c", subcore_axis_name="s",
        num_cores=NCORES, num_subcores=NSUB)

    def body(h0_hbm, h1_hbm, al_hbm, src_hbm, dst_hbm, out_hbm, s_hbm,
             als_v, ald_v, srcw, dstw, wj, ssb, rows0, rows1,
             out_sh, s_sh, sem0, sem1, sem2, sem3):
        cid = lax.axis_index("c")
        sid = lax.axis_index("s")
        off = sid * STRIPE

        zero16 = jnp.zeros((LANES,), jnp.float32)

        def zs(i, _):
            ssb[i, pl.ds(0, LANES)] = zero16
            return 0
        lax.fori_loop(0, CHUNK, zs, 0)
        done = 0
        while done < STRIPE:
            sz = min(CHUNK, STRIPE - done)
            pltpu.sync_copy(ssb.at[pl.ds(0, sz)],
                            s_sh.at[pl.ds(off + done, sz)])
            done += sz

        def zrow(i, _):
            for gidx in range(gv):
                rows0[i, pl.ds(gidx * LANES, LANES)] = zero16
                rows1[i, pl.ds(gidx * LANES, LANES)] = zero16
            return 0
        lax.fori_loop(0, CHUNK, zrow, 0)
        done = 0
        while done < STRIPE:
            sz = min(CHUNK, STRIPE - done)
            pltpu.sync_copy(rows0.at[pl.ds(0, sz)],
                            out_sh.at[pl.ds(off + done, sz)])
            done += sz

        pltpu.sync_copy(al_hbm.at[0], als_v)
        pltpu.sync_copy(al_hbm.at[1], ald_v)
        pltpu.sync_copy(src_hbm.at[sid], srcw)
        pltpu.sync_copy(dst_hbm.at[sid], dstw)

        plsc.subcore_barrier()

        lane = lax.iota(jnp.int32, LANES)
        base0 = sid * (nc * CHUNK)

        def edge_phase(h_hbm):
            def phase_a(j):
                base1 = base0 + j * CHUNK
                for i in range(CHUNK // LANES):
                    d16 = dstw[j, pl.ds(i * LANES, LANES)]
                    als = plsc.load_gather(
                        als_v, [srcw[j, pl.ds(i * LANES, LANES)]])
                    ald = plsc.load_gather(ald_v, [d16])
                    e = als + ald
                    e = jnp.where(e > 0.0, e, 0.2 * e)
                    w = jnp.exp(e)
                    geid = lax.broadcast(base1 + i * LANES, (LANES,)) + lane
                    w = jnp.where(geid < etot, w, 0.0)
                    wj[pl.ds(i * LANES, LANES)] = w
                    plsc.store_scatter(
                        ssb,
                        [lax.broadcast(i * LANES, (LANES,)) + lane,
                         jnp.zeros((LANES,), jnp.int32)], w)

                @pl.when(lax.rem(j, 2) == cid)
                def _():
                    pltpu.sync_copy(ssb, s_sh.at[dstw.at[j]], add=True)

            def multiply(rows):
                def mbody(i, _):
                    for u in range(4):
                        idx = i * 4 + u
                        ws = plsc.load_gather(
                            wj, [lax.broadcast(idx, (LANES,))])
                        for gidx in range(gv):
                            sl = pl.ds(gidx * LANES, LANES)
                            rows[idx, sl] = rows[idx, sl] * ws
                    return 0
                lax.fori_loop(0, CHUNK // 4, mbody, 0)

            def half_step(j, rows, g_sem, sc_sem, pf_j, pf_rows,
                          pf_g_sem, pf_sc_sem):
                pltpu.make_async_copy(
                    pf_rows, out_sh.at[dstw.at[0]], pf_sc_sem).wait()
                pltpu.async_copy(h_hbm.at[srcw.at[pf_j]], pf_rows, pf_g_sem)
                phase_a(j)
                pltpu.make_async_copy(
                    h_hbm.at[srcw.at[j]], rows, g_sem).wait()
                multiply(rows)
                pltpu.async_copy(rows, out_sh.at[dstw.at[j]], sc_sem,
                                 add=True)

            pltpu.async_copy(h_hbm.at[srcw.at[0]], rows0, sem0)
            pltpu.async_copy(rows1, out_sh.at[dstw.at[0]], sem3, add=True)

            def pair_body(k, _):
                j0 = 2 * k
                j1 = j0 + 1
                j2 = jnp.where(j1 + 1 < nc, j1 + 1, 0)
                half_step(j0, rows0, sem0, sem2, j1, rows1, sem1, sem3)
                half_step(j1, rows1, sem1, sem3, j2, rows0, sem0, sem2)
                return 0
            lax.fori_loop(0, nc // 2, pair_body, 0)
            pltpu.make_async_copy(
                h_hbm.at[srcw.at[0]], rows0, sem0).wait()
            pltpu.make_async_copy(
                rows1, out_sh.at[dstw.at[0]], sem3).wait()

        @pl.when(cid == 0)
        def _():
            edge_phase(h0_hbm)

        @pl.when(cid == 1)
        def _():
            edge_phase(h1_hbm)

        plsc.subcore_barrier()

        pltpu.sync_copy(s_sh.at[pl.ds(off, STRIPE)],
                        s_hbm.at[cid, pl.ds(off, STRIPE)])

        pltpu.sync_copy(out_sh.at[pl.ds(off, STRIPE)],
                        out_hbm.at[cid, pl.ds(off, STRIPE)])

    kern = pl.kernel(
        body,
        out_type=[
            jax.ShapeDtypeStruct((NCORES, N, hw), jnp.float32),
            jax.ShapeDtypeStruct((NCORES, N, LANES), jnp.float32),
        ],
        mesh=mesh,
        compiler_params=pltpu.CompilerParams(
            needs_layout_passes=False, use_tc_tiling_on_sc=False),
        scratch_types=[
            pltpu.VMEM((AL_PAD,), jnp.float32),
            pltpu.VMEM((AL_PAD,), jnp.float32),
            pltpu.VMEM((nc, CHUNK), jnp.int32),
            pltpu.VMEM((nc, CHUNK), jnp.int32),
            pltpu.VMEM((CHUNK,), jnp.float32),
            pltpu.VMEM((CHUNK, LANES), jnp.float32),
            pltpu.VMEM((CHUNK, hw), jnp.float32),
            pltpu.VMEM((CHUNK, hw), jnp.float32),
            pltpu.VMEM_SHARED((N, hw), jnp.float32),
            pltpu.VMEM_SHARED((N, LANES), jnp.float32),
            pltpu.SemaphoreType.DMA,
            pltpu.SemaphoreType.DMA,
            pltpu.SemaphoreType.DMA,
            pltpu.SemaphoreType.DMA,
        ],
    )
    return kern(h0, h1, al, src3d, dst3d)


def kernel(x, edge_index, W1, as1, ad1, b1, g1, be1,
           W2, as2, ad2, b2, g2, be2, W3, as3, ad3, b3, g3, be3):
    loops = jnp.arange(N, dtype=jnp.int32)
    E = edge_index.shape[1]
    etot = E + N
    nc = -(-etot // (NSUB * CHUNK))
    epad = NSUB * nc * CHUNK
    pad = epad - etot
    src = jnp.concatenate(
        [edge_index[0], loops, jnp.zeros((pad,), jnp.int32)])
    dst = jnp.concatenate(
        [edge_index[1], loops, jnp.zeros((pad,), jnp.int32)])
    src3d = src.reshape(NSUB, nc, CHUNK)
    dst3d = dst.reshape(NSUB, nc, CHUNK)

    h0, h1, al = _matmul_attn(x, W1, as1, ad1)
    parts, s_parts = _sc_gat(h0, h1, al, src3d, dst3d, nc, etot)
    for (bp, gp, bep, W, a_s, a_d) in (
            (b1, g1, be1, W2, as2, ad2),
            (b2, g2, be2, W3, as3, ad3)):
        h0, h1, al = _combine_matmul_attn(
            parts, s_parts, bp, gp, bep, W, a_s, a_d)
        parts, s_parts = _sc_gat(h0, h1, al, src3d, dst3d, nc, etot)
    return _combine_bn(parts, s_parts, b3, g3, be3)


# speedup vs baseline: 1.0382x; 1.0285x over previous
import jax
import jax.numpy as jnp
from jax import lax
from jax.experimental import pallas as pl
from jax.experimental.pallas import tpu as pltpu
from jax.experimental.pallas import tpu_sc as plsc

N = 10000
LANES = 16
CHUNK = 128
NCORES = 2
NSUB = 16
STRIPE = N // NSUB
S_PAD = 10016
AL_PAD = 10008


def _matmul_attn(z, W, a_s, a_d):
    n, _ = z.shape
    c = W.shape[1]
    hw = c // 2

    def body(z_ref, w_ref, as_ref, ad_ref, h0_ref, h1_ref, al_ref):
        h = jnp.dot(z_ref[...].astype(jnp.bfloat16),
                    w_ref[...].astype(jnp.bfloat16),
                    preferred_element_type=jnp.float32)
        h0_ref[...] = h[:, :hw]
        h1_ref[...] = h[:, hw:]
        hb = h.astype(jnp.bfloat16)
        al_s = jnp.dot(hb, as_ref[0].astype(jnp.bfloat16),
                       preferred_element_type=jnp.float32)
        al_d = jnp.dot(hb, ad_ref[0].astype(jnp.bfloat16),
                       preferred_element_type=jnp.float32)
        al = jnp.stack([al_s, al_d])
        al_ref[...] = jnp.concatenate(
            [al, jnp.zeros((2, AL_PAD - n), jnp.float32)], axis=1)

    return pl.pallas_call(
        body,
        out_shape=[
            jax.ShapeDtypeStruct((n, hw), jnp.float32),
            jax.ShapeDtypeStruct((n, hw), jnp.float32),
            jax.ShapeDtypeStruct((2, AL_PAD), jnp.float32),
        ],
    )(z, W, a_s.reshape(1, c), a_d.reshape(1, c))


def _combine_bn(parts, s_parts, b, g, be):
    c = b.shape[0]

    def body(p_ref, s_ref, b_ref, g_ref, be_ref, z_ref):
        hsum = p_ref[0] + p_ref[1]
        s = jnp.sum(s_ref[0] + s_ref[1], axis=1, keepdims=True)
        z = hsum / (s + 1e-16) + b_ref[0][None, :]
        mean = jnp.mean(z, axis=0, keepdims=True)
        var = jnp.mean((z - mean) ** 2, axis=0, keepdims=True)
        xn = (z - mean) * lax.rsqrt(var + 1e-5)
        z_ref[...] = jnp.maximum(xn * g_ref[0][None, :] + be_ref[0][None, :], 0.0)

    return pl.pallas_call(
        body,
        out_shape=jax.ShapeDtypeStruct((N, c), jnp.float32),
    )(parts, s_parts, b.reshape(1, c), g.reshape(1, c), be.reshape(1, c))


def _combine_matmul_attn(parts, s_parts, b, g, be, W, a_s, a_d, concat):
    cp = b.shape[0]
    c = W.shape[1]

    def body(p_ref, s_ref, b_ref, g_ref, be_ref, w_ref, as_ref, ad_ref,
             h_ref, al_ref):
        if concat:
            hsum = jnp.concatenate([p_ref[0], p_ref[1]], axis=1)
        else:
            hsum = p_ref[0] + p_ref[1]
        s = jnp.sum(s_ref[0] + s_ref[1], axis=1, keepdims=True)
        z = hsum / (s + 1e-16) + b_ref[0][None, :]
        mean = jnp.mean(z, axis=0, keepdims=True)
        var = jnp.mean((z - mean) ** 2, axis=0, keepdims=True)
        xn = (z - mean) * lax.rsqrt(var + 1e-5)
        z = jnp.maximum(xn * g_ref[0][None, :] + be_ref[0][None, :], 0.0)
        h = jnp.dot(z.astype(jnp.bfloat16), w_ref[...].astype(jnp.bfloat16),
                    preferred_element_type=jnp.float32)
        h_ref[...] = h
        hb = h.astype(jnp.bfloat16)
        al_s = jnp.dot(hb, as_ref[0].astype(jnp.bfloat16),
                       preferred_element_type=jnp.float32)
        al_d = jnp.dot(hb, ad_ref[0].astype(jnp.bfloat16),
                       preferred_element_type=jnp.float32)
        al = jnp.stack([al_s, al_d])
        al_ref[...] = jnp.concatenate(
            [al, jnp.zeros((2, AL_PAD - N), jnp.float32)], axis=1)

    return pl.pallas_call(
        body,
        out_shape=[
            jax.ShapeDtypeStruct((N, c), jnp.float32),
            jax.ShapeDtypeStruct((2, AL_PAD), jnp.float32),
        ],
    )(parts, s_parts, b.reshape(1, cp), g.reshape(1, cp), be.reshape(1, cp),
      W, a_s.reshape(1, c), a_d.reshape(1, c))


def _sc_gat(h0, h1, al, src3d, dst3d, nc, etot, esplit):
    hw = h0.shape[1]
    gv = hw // LANES
    mesh = plsc.VectorSubcoreMesh(
        core_axis_name="c", subcore_axis_name="s",
        num_cores=NCORES, num_subcores=NSUB)

    def body(h0_hbm, h1_hbm, al_hbm, src_hbm, dst_hbm, out_hbm, s_hbm,
             als_v, ald_v, srcw, dstw, wj, ssb, rows0, rows1,
             out_sh, s_sh, sem0, sem1, sem2, sem3):
        cid = lax.axis_index("c")
        sid = lax.axis_index("s")
        wid = cid * NSUB + sid if esplit else sid
        off = sid * STRIPE

        zero16 = jnp.zeros((LANES,), jnp.float32)

        def zs(i, _):
            ssb[i, pl.ds(0, LANES)] = zero16
            return 0
        lax.fori_loop(0, CHUNK, zs, 0)
        done = 0
        while done < STRIPE:
            sz = min(CHUNK, STRIPE - done)
            pltpu.sync_copy(ssb.at[pl.ds(0, sz)],
                            s_sh.at[pl.ds(off + done, sz)])
            done += sz

        def zrow(i, _):
            for gidx in range(gv):
                rows0[i, pl.ds(gidx * LANES, LANES)] = zero16
                rows1[i, pl.ds(gidx * LANES, LANES)] = zero16
            return 0
        lax.fori_loop(0, CHUNK, zrow, 0)
        done = 0
        while done < STRIPE:
            sz = min(CHUNK, STRIPE - done)
            pltpu.sync_copy(rows0.at[pl.ds(0, sz)],
                            out_sh.at[pl.ds(off + done, sz)])
            done += sz

        pltpu.sync_copy(al_hbm.at[0], als_v)
        pltpu.sync_copy(al_hbm.at[1], ald_v)
        pltpu.sync_copy(src_hbm.at[wid], srcw)
        pltpu.sync_copy(dst_hbm.at[wid], dstw)

        plsc.subcore_barrier()

        lane = lax.iota(jnp.int32, LANES)
        base0 = wid * (nc * CHUNK)

        def edge_phase(h_hbm):
            def phase_a(j):
                base1 = base0 + j * CHUNK
                for i in range(CHUNK // LANES):
                    d16 = dstw[j, pl.ds(i * LANES, LANES)]
                    als = plsc.load_gather(
                        als_v, [srcw[j, pl.ds(i * LANES, LANES)]])
                    ald = plsc.load_gather(ald_v, [d16])
                    e = als + ald
                    e = jnp.where(e > 0.0, e, 0.2 * e)
                    w = jnp.exp(e)
                    geid = lax.broadcast(base1 + i * LANES, (LANES,)) + lane
                    w = jnp.where(geid < etot, w, 0.0)
                    wj[pl.ds(i * LANES, LANES)] = w
                    plsc.store_scatter(
                        ssb,
                        [lax.broadcast(i * LANES, (LANES,)) + lane,
                         jnp.zeros((LANES,), jnp.int32)], w)

                if esplit:
                    pltpu.sync_copy(ssb, s_sh.at[dstw.at[j]], add=True)
                else:
                    @pl.when(lax.rem(j, 2) == cid)
                    def _():
                        pltpu.sync_copy(ssb, s_sh.at[dstw.at[j]], add=True)

            def multiply(rows):
                def mbody(i, _):
                    for u in range(4):
                        idx = i * 4 + u
                        ws = plsc.load_gather(
                            wj, [lax.broadcast(idx, (LANES,))])
                        for gidx in range(gv):
                            sl = pl.ds(gidx * LANES, LANES)
                            rows[idx, sl] = rows[idx, sl] * ws
                    return 0
                lax.fori_loop(0, CHUNK // 4, mbody, 0)

            def half_step(j, rows, g_sem, sc_sem, pf_j, pf_rows,
                          pf_g_sem, pf_sc_sem):
                pltpu.make_async_copy(
                    pf_rows, out_sh.at[dstw.at[0]], pf_sc_sem).wait()
                pltpu.async_copy(h_hbm.at[srcw.at[pf_j]], pf_rows, pf_g_sem)
                phase_a(j)
                pltpu.make_async_copy(
                    h_hbm.at[srcw.at[j]], rows, g_sem).wait()
                multiply(rows)
                pltpu.async_copy(rows, out_sh.at[dstw.at[j]], sc_sem,
                                 add=True)

            pltpu.async_copy(h_hbm.at[srcw.at[0]], rows0, sem0)
            pltpu.async_copy(rows1, out_sh.at[dstw.at[0]], sem3, add=True)

            def pair_body(k, _):
                j0 = 2 * k
                j1 = j0 + 1
                j2 = jnp.where(j1 + 1 < nc, j1 + 1, 0)
                half_step(j0, rows0, sem0, sem2, j1, rows1, sem1, sem3)
                half_step(j1, rows1, sem1, sem3, j2, rows0, sem0, sem2)
                return 0
            lax.fori_loop(0, nc // 2, pair_body, 0)
            pltpu.make_async_copy(
                h_hbm.at[srcw.at[0]], rows0, sem0).wait()
            pltpu.make_async_copy(
                rows1, out_sh.at[dstw.at[0]], sem3).wait()

        if esplit:
            edge_phase(h0_hbm)
        else:
            @pl.when(cid == 0)
            def _():
                edge_phase(h0_hbm)

            @pl.when(cid == 1)
            def _():
                edge_phase(h1_hbm)

        plsc.subcore_barrier()

        pltpu.sync_copy(s_sh.at[pl.ds(off, STRIPE)],
                        s_hbm.at[cid, pl.ds(off, STRIPE)])

        pltpu.sync_copy(out_sh.at[pl.ds(off, STRIPE)],
                        out_hbm.at[cid, pl.ds(off, STRIPE)])

    kern = pl.kernel(
        body,
        out_type=[
            jax.ShapeDtypeStruct((NCORES, N, hw), jnp.float32),
            jax.ShapeDtypeStruct((NCORES, N, LANES), jnp.float32),
        ],
        mesh=mesh,
        compiler_params=pltpu.CompilerParams(
            needs_layout_passes=False, use_tc_tiling_on_sc=False),
        scratch_types=[
            pltpu.VMEM((AL_PAD,), jnp.float32),
            pltpu.VMEM((AL_PAD,), jnp.float32),
            pltpu.VMEM((nc, CHUNK), jnp.int32),
            pltpu.VMEM((nc, CHUNK), jnp.int32),
            pltpu.VMEM((CHUNK,), jnp.float32),
            pltpu.VMEM((CHUNK, LANES), jnp.float32),
            pltpu.VMEM((CHUNK, hw), jnp.float32),
            pltpu.VMEM((CHUNK, hw), jnp.float32),
            pltpu.VMEM_SHARED((N, hw), jnp.float32),
            pltpu.VMEM_SHARED((N, LANES), jnp.float32),
            pltpu.SemaphoreType.DMA,
            pltpu.SemaphoreType.DMA,
            pltpu.SemaphoreType.DMA,
            pltpu.SemaphoreType.DMA,
        ],
    )
    return kern(h0, h1, al, src3d, dst3d)


def kernel(x, edge_index, W1, as1, ad1, b1, g1, be1,
           W2, as2, ad2, b2, g2, be2, W3, as3, ad3, b3, g3, be3):
    loops = jnp.arange(N, dtype=jnp.int32)
    E = edge_index.shape[1]
    etot = E + N

    def padded(nworkers):
        nc = -(-etot // (nworkers * CHUNK))
        if nc % 2:
            nc += 1
        pad = nworkers * nc * CHUNK - etot
        s = jnp.concatenate(
            [edge_index[0], loops, jnp.zeros((pad,), jnp.int32)])
        d = jnp.concatenate(
            [edge_index[1], loops, jnp.zeros((pad,), jnp.int32)])
        return (s.reshape(nworkers, nc, CHUNK),
                d.reshape(nworkers, nc, CHUNK), nc)

    src16, dst16, nc16 = padded(NSUB)
    src32, dst32, nc32 = padded(NCORES * NSUB)

    h0, h1, al = _matmul_attn(x, W1, as1, ad1)
    parts, s_parts = _sc_gat(h0, h1, al, src16, dst16, nc16, etot, False)
    h, al = _combine_matmul_attn(
        parts, s_parts, b1, g1, be1, W2, as2, ad2, True)
    parts, s_parts = _sc_gat(h, h, al, src32, dst32, nc32, etot, True)
    h, al = _combine_matmul_attn(
        parts, s_parts, b2, g2, be2, W3, as3, ad3, False)
    parts, s_parts = _sc_gat(h, h, al, src32, dst32, nc32, etot, True)
    return _combine_bn(parts, s_parts, b3, g3, be3)
